# baseline (speedup 1.0000x reference)
import jax
import jax.numpy as jnp
from jax.experimental import pallas as pl
from jax.experimental.pallas import tpu as pltpu

TOPK = 10
_NEG = -3.0e38
_TN = 2048


def _bdot(x, y, dims):
    return jax.lax.dot_general(
        x, y, (dims, ((), ())),
        preferred_element_type=jnp.float32,
        precision=jax.lax.Precision.DEFAULT)


def _fused_body(h_ref, w1_ref, b1_ref, wr_ref,
                wa_ref, ba_ref, wb_ref, bb_ref, wc_ref, bc_ref, br_ref,
                lr_ref, rk_ref, ta_ref,
                acc_scr, s_scr, h0_scr, rows_scr):
    i = pl.program_id(0)
    k = pl.program_id(1)
    ngrid = pl.num_programs(0)

    part = _bdot(h_ref[...], w1_ref[...], ((1,), (1,)))

    @pl.when(k == 0)
    def _store_partial():
        acc_scr[...] = part

    @pl.when(k == 1)
    def _complete_tile():
        h0 = jnp.maximum(acc_scr[...] + part + b1_ref[...], 0.0)
        h0_scr[pl.ds(i * _TN, _TN), :] = h0
        s_scr[pl.ds(i, 1), :] = _bdot(wr_ref[...], h0, ((1,), (1,)))

    @pl.when((i == ngrid - 1) & (k == 1))
    def _finish():
        s = s_scr[...]
        rows, cols = s.shape
        flat_id = (jax.lax.broadcasted_iota(jnp.int32, (rows, cols), 0) * cols
                   + jax.lax.broadcasted_iota(jnp.int32, (rows, cols), 1))

        sm = s
        for j in range(TOPK):
            m = jnp.max(sm)
            idx = jnp.min(jnp.where(sm == m, flat_id, jnp.int32(2**31 - 1)))
            rows_scr[pl.ds(j, 1), :] = h0_scr[pl.ds(idx, 1), :]
            sm = jnp.where(flat_id == idx, _NEG, sm)

        h0t = rows_scr[0:TOPK, :]

        a = jnp.tanh(_bdot(h0t, wa_ref[...], ((1,), (1,))) + ba_ref[...])
        g = jax.nn.sigmoid(_bdot(h0t, wb_ref[...], ((1,), (1,))) + bb_ref[...])
        att = _bdot(wc_ref[...], a * g, ((1,), (1,))) + bc_ref[0, 0]

        e = jnp.exp(att - jnp.max(att))
        w = e / jnp.sum(e)
        ta_ref[...] = w

        m_vec = _bdot(w, h0t, ((1,), (0,)))
        m16 = m_vec.astype(jnp.bfloat16).astype(jnp.float32)
        w16 = wr_ref[...].astype(jnp.bfloat16).astype(jnp.float32)
        lr_val = jnp.sum(m16 * w16) + br_ref[0, 0]
        lr_ref[...] = jnp.full((1, 1), lr_val, jnp.float32)
        rk_ref[...] = jnp.full((1, 1), jnp.exp(lr_val), jnp.float32)


@jax.jit
def kernel(h, W1, b1, Wa, ba, Wb, bb, Wc, bc, Wr, br):
    N, E = h.shape
    H = W1.shape[0]
    E2 = E // 2
    grid = N // _TN

    b1r = b1.reshape(1, H)
    wrr = Wr.reshape(1, H)
    brr = br.reshape(1, 1)
    bar = ba.reshape(1, -1)
    bbr = bb.reshape(1, -1)
    bcr = bc.reshape(1, 1)

    lr, rk, ta = pl.pallas_call(
        _fused_body,
        grid=(grid, 2),
        in_specs=[
            pl.BlockSpec((_TN, E2), lambda i, k: (i, k)),
            pl.BlockSpec((H, E2), lambda i, k: (0, k)),
            pl.BlockSpec((1, H), lambda i, k: (0, 0)),
            pl.BlockSpec((1, H), lambda i, k: (0, 0)),
            pl.BlockSpec((Wa.shape[0], H), lambda i, k: (0, 0)),
            pl.BlockSpec((1, Wa.shape[0]), lambda i, k: (0, 0)),
            pl.BlockSpec((Wb.shape[0], H), lambda i, k: (0, 0)),
            pl.BlockSpec((1, Wb.shape[0]), lambda i, k: (0, 0)),
            pl.BlockSpec((1, Wa.shape[0]), lambda i, k: (0, 0)),
            pl.BlockSpec((1, 1), lambda i, k: (0, 0)),
            pl.BlockSpec((1, 1), lambda i, k: (0, 0)),
        ],
        out_specs=(
            pl.BlockSpec((1, 1), lambda i, k: (0, 0)),
            pl.BlockSpec((1, 1), lambda i, k: (0, 0)),
            pl.BlockSpec((1, TOPK), lambda i, k: (0, 0)),
        ),
        out_shape=(
            jax.ShapeDtypeStruct((1, 1), jnp.float32),
            jax.ShapeDtypeStruct((1, 1), jnp.float32),
            jax.ShapeDtypeStruct((1, TOPK), jnp.float32),
        ),
        scratch_shapes=[
            pltpu.VMEM((_TN, H), jnp.float32),
            pltpu.VMEM((grid, _TN), jnp.float32),
            pltpu.VMEM((N, H), jnp.float32),
            pltpu.VMEM((16, H), jnp.float32),
        ],
    )(h, W1, b1r, wrr, Wa, bar, Wb, bbr, Wc, bcr, brr)

    return lr, rk, ta
